# Initial kernel scaffold; baseline (speedup 1.0000x reference)
import functools

import jax
import jax.numpy as jnp
from jax import lax
from jax.experimental import pallas as pl
from jax.experimental.pallas import tpu as pltpu
from jax.experimental.pallas import tpu_sc as plsc

N = 50000
E = 800000
NC = 2
NS = 16
NW = NC * NS
CH = 128
CHUNKS_W = 196
EP = NW * CH * CHUNKS_W
N_PAD = 50176
ZROWS = 392
BN = 2000
NBLK = N // BN


def _segsum(src2d, dst2d, feat, d):
  mesh = plsc.VectorSubcoreMesh(
      core_axis_name="c", subcore_axis_name="s", num_cores=NC, num_subcores=NS)

  @functools.partial(
      pl.kernel,
      out_type=jax.ShapeDtypeStruct((NC, N, d), jnp.float32),
      mesh=mesh,
      scratch_types=[
          pltpu.VMEM((CHUNKS_W, CH), jnp.int32),
          pltpu.VMEM((CHUNKS_W, CH), jnp.int32),
          pltpu.VMEM((CH, d), jnp.float32),
          pltpu.VMEM((ZROWS, d), jnp.float32),
          pltpu.VMEM_SHARED((N_PAD, d), jnp.float32),
          pltpu.SemaphoreType.DMA,
      ],
  )
  def seg(src_hbm, dst_hbm, feat_hbm, out_hbm, sidx, didx, rows, zbuf, acc,
          sem):
    c = lax.axis_index("c")
    s = lax.axis_index("s")
    wid = c * NS + s

    pltpu.sync_copy(src_hbm.at[pl.ds(wid * CHUNKS_W, CHUNKS_W)], sidx)
    pltpu.sync_copy(dst_hbm.at[pl.ds(wid * CHUNKS_W, CHUNKS_W)], didx)

    z16 = jnp.zeros((16,), jnp.float32)

    def zrow(i, carry):
      for g in range(d // 16):
        zbuf[i, pl.ds(g * 16, 16)] = z16
      return carry

    lax.fori_loop(0, ZROWS, zrow, 0)
    zbase = s * (N_PAD // NS)

    def zcopy(j, carry):
      pltpu.sync_copy(zbuf, acc.at[pl.ds(zbase + j * ZROWS, ZROWS)])
      return carry

    lax.fori_loop(0, (N_PAD // NS) // ZROWS, zcopy, 0)
    plsc.subcore_barrier()

    def body(j, carry):
      pltpu.async_copy(feat_hbm.at[sidx.at[j]], rows, sem).wait()
      pltpu.sync_copy(rows, acc.at[didx.at[j]], add=True)
      return carry

    lax.fori_loop(0, CHUNKS_W, body, 0)
    plsc.subcore_barrier()

    wbase = s * (N // NS)
    pltpu.sync_copy(acc.at[pl.ds(wbase, N // NS)],
                    out_hbm.at[c, pl.ds(wbase, N // NS)])

  return seg(src2d, dst2d, feat)


def _dot_t(x, w):
  return lax.dot_general(x, w, (((1,), (1,)), ((), ())),
                         preferred_element_type=jnp.float32)


def _proj2(z, w_rel, w_root):
  d_in = z.shape[1]
  d_out = w_rel.shape[0]

  def body(z_ref, wr_ref, wt_ref, m_ref, r_ref):
    zb = z_ref[...]
    m_ref[...] = _dot_t(zb, wr_ref[...])
    r_ref[...] = _dot_t(zb, wt_ref[...])

  sds = jax.ShapeDtypeStruct((N, d_out), jnp.float32)
  return pl.pallas_call(
      body,
      grid=(NBLK,),
      in_specs=[
          pl.BlockSpec((BN, d_in), lambda i: (i, 0)),
          pl.BlockSpec((d_out, d_in), lambda i: (0, 0)),
          pl.BlockSpec((d_out, d_in), lambda i: (0, 0)),
      ],
      out_specs=[
          pl.BlockSpec((BN, d_out), lambda i: (i, 0)),
          pl.BlockSpec((BN, d_out), lambda i: (i, 0)),
      ],
      out_shape=(sds, sds),
  )(z, w_rel, w_root)


def _combine_proj(p, r, b, w_rel, w_root):
  d = r.shape[1]
  d_out = w_rel.shape[0]

  def body(p_ref, r_ref, b_ref, wr_ref, wt_ref, m_ref, r2_ref):
    pb = p_ref[...]
    h = jnp.maximum(pb[0] + pb[1] + r_ref[...] + b_ref[...][None, :], 0.0)
    m_ref[...] = _dot_t(h, wr_ref[...])
    r2_ref[...] = _dot_t(h, wt_ref[...])

  sds = jax.ShapeDtypeStruct((N, d_out), jnp.float32)
  return pl.pallas_call(
      body,
      grid=(NBLK,),
      in_specs=[
          pl.BlockSpec((2, BN, d), lambda i: (0, i, 0)),
          pl.BlockSpec((BN, d), lambda i: (i, 0)),
          pl.BlockSpec((d,), lambda i: (0,)),
          pl.BlockSpec((d_out, d), lambda i: (0, 0)),
          pl.BlockSpec((d_out, d), lambda i: (0, 0)),
      ],
      out_specs=[
          pl.BlockSpec((BN, d_out), lambda i: (i, 0)),
          pl.BlockSpec((BN, d_out), lambda i: (i, 0)),
      ],
      out_shape=(sds, sds),
  )(p, r, b, w_rel, w_root)


def _combine_relu(p, r, b):
  d = r.shape[1]

  def body(p_ref, r_ref, b_ref, h_ref):
    pb = p_ref[...]
    h_ref[...] = jnp.maximum(pb[0] + pb[1] + r_ref[...] + b_ref[...][None, :],
                             0.0)

  return pl.pallas_call(
      body,
      grid=(NBLK,),
      in_specs=[
          pl.BlockSpec((2, BN, d), lambda i: (0, i, 0)),
          pl.BlockSpec((BN, d), lambda i: (i, 0)),
          pl.BlockSpec((d,), lambda i: (0,)),
      ],
      out_specs=pl.BlockSpec((BN, d), lambda i: (i, 0)),
      out_shape=jax.ShapeDtypeStruct((N, d), jnp.float32),
  )(p, r, b)


def _final(p, h2, b3, w_rel, w_root):
  d = h2.shape[1]

  def body(p_ref, h_ref, b_ref, wr_ref, wt_ref, o_ref):
    pb = p_ref[...]
    agg = pb[0] + pb[1]
    logits = (_dot_t(agg, wr_ref[...]) + b_ref[...][None, :]
              + _dot_t(h_ref[...], wt_ref[...]))
    mx = jnp.max(logits, axis=-1, keepdims=True)
    ex = jnp.exp(logits - mx)
    o_ref[...] = ex / jnp.sum(ex, axis=-1, keepdims=True)

  return pl.pallas_call(
      body,
      grid=(NBLK,),
      in_specs=[
          pl.BlockSpec((2, BN, d), lambda i: (0, i, 0)),
          pl.BlockSpec((BN, d), lambda i: (i, 0)),
          pl.BlockSpec((2,), lambda i: (0,)),
          pl.BlockSpec((2, d), lambda i: (0, 0)),
          pl.BlockSpec((2, d), lambda i: (0, 0)),
      ],
      out_specs=pl.BlockSpec((BN, 2), lambda i: (i, 0)),
      out_shape=jax.ShapeDtypeStruct((N, 2), jnp.float32),
  )(p, h2, b3, w_rel, w_root)


def kernel(z, edge_index, W1_rel, b1, W1_root, W2_rel, b2, W2_root,
           W3_rel, b3, W3_root):
  pad = EP - E
  src = jnp.concatenate(
      [edge_index[0], jnp.zeros((pad,), jnp.int32)]).reshape(-1, CH)
  dst = jnp.concatenate(
      [edge_index[1], jnp.full((pad,), N, jnp.int32)]).reshape(-1, CH)

  m1, r1 = _proj2(z, W1_rel, W1_root)
  p1 = _segsum(src, dst, m1, 32)
  m2, r2 = _combine_proj(p1, r1, b1, W2_rel, W2_root)
  p2 = _segsum(src, dst, m2, 16)
  h2 = _combine_relu(p2, r2, b2)
  p3 = _segsum(src, dst, h2, 16)
  return _final(p3, h2, b3, W3_rel, W3_root)


# speedup vs baseline: 6.4605x; 6.4605x over previous
import functools

import jax
import jax.numpy as jnp
from jax import lax
from jax.experimental import pallas as pl
from jax.experimental.pallas import tpu as pltpu
from jax.experimental.pallas import tpu_sc as plsc

N = 50000
E = 800000
NC = 2
NS = 16
NW = NC * NS
CH = 128
CHUNKS_W = 200
EP = NW * CH * CHUNKS_W
CHUNKS_T = EP // (CH * NS)
N_PAD = 50176
G = 40
BN = 2000
NBLK = N // BN


def _seg_body(feat_hbm, src_hbm, dst_hbm, sidx, didx, bufs, acc, sem,
              chunk0, nchunks):
  def group(gi, carry):
    base = chunk0 + gi * G
    pltpu.sync_copy(src_hbm.at[pl.ds(base, G)], sidx)
    pltpu.sync_copy(dst_hbm.at[pl.ds(base, G)], didx)

    def body(j, carry2):
      for t in range(2):
        jj = 2 * j + t
        pltpu.async_copy(feat_hbm.at[sidx.at[jj]], bufs.at[t], sem).wait()
        pltpu.sync_copy(bufs.at[t], acc.at[didx.at[jj]], add=True)
      return carry2

    return lax.fori_loop(0, G // 2, body, carry)

  lax.fori_loop(0, nchunks // G, group, 0)


def _zero_acc(rows, acc, s, d):
  z16 = jnp.zeros((16,), jnp.float32)

  def zrow(i, carry):
    for g in range(d // 16):
      rows[i, pl.ds(g * 16, 16)] = z16
    return carry

  lax.fori_loop(0, CH, zrow, 0)
  zbase = s * (N_PAD // NS)

  def zcopy(j, carry):
    pltpu.sync_copy(rows, acc.at[pl.ds(zbase + j * CH, CH)])
    return carry

  lax.fori_loop(0, (N_PAD // NS) // CH, zcopy, 0)
  if (N_PAD // NS) % CH:
    pltpu.sync_copy(rows, acc.at[pl.ds(zbase + (N_PAD // NS) - CH, CH)])


def _writeout(acc, out_hbm, c, s):
  wrows = N_PAD // NS
  wbase = s * wrows
  pltpu.sync_copy(acc.at[pl.ds(wbase, wrows)],
                  out_hbm.at[c, pl.ds(wbase, wrows)])


def _scratch(d):
  return [
      pltpu.VMEM((G, CH), jnp.int32),
      pltpu.VMEM((G, CH), jnp.int32),
      pltpu.VMEM((2, CH, d), jnp.float32),
      pltpu.VMEM_SHARED((N_PAD, d), jnp.float32),
      pltpu.SemaphoreType.DMA,
  ]


def _mesh():
  return plsc.VectorSubcoreMesh(
      core_axis_name="c", subcore_axis_name="s", num_cores=NC, num_subcores=NS)


def _segsum_cols(src2d, dst2d, feat_lo, feat_hi, d):
  @functools.partial(
      pl.kernel,
      out_type=jax.ShapeDtypeStruct((NC, N_PAD, d), jnp.float32),
      mesh=_mesh(),
      compiler_params=pltpu.CompilerParams(use_tc_tiling_on_sc=False),
      scratch_types=_scratch(d),
  )
  def seg(src_hbm, dst_hbm, lo_hbm, hi_hbm, out_hbm, sidx, didx, bufs, acc,
          sem):
    c = lax.axis_index("c")
    s = lax.axis_index("s")
    _zero_acc(bufs.at[0], acc, s, d)
    plsc.subcore_barrier()

    @pl.when(c == 0)
    def _():
      _seg_body(lo_hbm, src_hbm, dst_hbm, sidx, didx, bufs, acc, sem,
                s * CHUNKS_T, CHUNKS_T)

    @pl.when(c == 1)
    def _():
      _seg_body(hi_hbm, src_hbm, dst_hbm, sidx, didx, bufs, acc, sem,
                s * CHUNKS_T, CHUNKS_T)

    plsc.subcore_barrier()
    _writeout(acc, out_hbm, c, s)

  return seg(src2d, dst2d, feat_lo, feat_hi)


def _segsum_edges(src2d, dst2d, feat, d):
  @functools.partial(
      pl.kernel,
      out_type=jax.ShapeDtypeStruct((NC, N_PAD, d), jnp.float32),
      mesh=_mesh(),
      compiler_params=pltpu.CompilerParams(use_tc_tiling_on_sc=False),
      scratch_types=_scratch(d),
  )
  def seg(src_hbm, dst_hbm, feat_hbm, out_hbm, sidx, didx, bufs, acc, sem):
    c = lax.axis_index("c")
    s = lax.axis_index("s")
    _zero_acc(bufs.at[0], acc, s, d)
    plsc.subcore_barrier()
    wid = c * NS + s
    _seg_body(feat_hbm, src_hbm, dst_hbm, sidx, didx, bufs, acc, sem,
              wid * CHUNKS_W, CHUNKS_W)
    plsc.subcore_barrier()
    _writeout(acc, out_hbm, c, s)

  return seg(src2d, dst2d, feat)


def _dot_t(x, w):
  return lax.dot_general(x, w, (((1,), (1,)), ((), ())),
                         preferred_element_type=jnp.float32)


def _split(z):
  def body(z_ref, lo_ref, hi_ref):
    zb = z_ref[...]
    lo_ref[...] = zb[:, :32]
    hi_ref[...] = zb[:, 32:]

  sds = jax.ShapeDtypeStruct((N, 32), jnp.float32)
  return pl.pallas_call(
      body,
      grid=(NBLK,),
      in_specs=[pl.BlockSpec((BN, 64), lambda i: (i, 0))],
      out_specs=[
          pl.BlockSpec((BN, 32), lambda i: (i, 0)),
          pl.BlockSpec((BN, 32), lambda i: (i, 0)),
      ],
      out_shape=(sds, sds),
  )(z)


def _layer1(p1, z, w_rel, b, w_root):
  def body(p_ref, z_ref, wr_ref, b_ref, wt_ref, h_ref):
    pb = p_ref[...]
    wr = wr_ref[...]
    agg_dot = _dot_t(pb[0], wr[:, :32]) + _dot_t(pb[1], wr[:, 32:])
    h_ref[...] = jnp.maximum(
        agg_dot + b_ref[...][None, :] + _dot_t(z_ref[...], wt_ref[...]), 0.0)

  return pl.pallas_call(
      body,
      grid=(NBLK,),
      in_specs=[
          pl.BlockSpec((2, BN, 32), lambda i: (0, i, 0)),
          pl.BlockSpec((BN, 64), lambda i: (i, 0)),
          pl.BlockSpec((32, 64), lambda i: (0, 0)),
          pl.BlockSpec((32,), lambda i: (0,)),
          pl.BlockSpec((32, 64), lambda i: (0, 0)),
      ],
      out_specs=pl.BlockSpec((BN, 32), lambda i: (i, 0)),
      out_shape=jax.ShapeDtypeStruct((N, 32), jnp.float32),
  )(p1, z, w_rel, b, w_root)


def _layer2(p2, h1, w_rel, b, w_root):
  def body(p_ref, h_ref, wr_ref, b_ref, wt_ref, o_ref):
    pb = p_ref[...]
    agg = pb[0] + pb[1]
    o_ref[...] = jnp.maximum(
        _dot_t(agg, wr_ref[...]) + b_ref[...][None, :]
        + _dot_t(h_ref[...], wt_ref[...]), 0.0)

  return pl.pallas_call(
      body,
      grid=(NBLK,),
      in_specs=[
          pl.BlockSpec((2, BN, 32), lambda i: (0, i, 0)),
          pl.BlockSpec((BN, 32), lambda i: (i, 0)),
          pl.BlockSpec((16, 32), lambda i: (0, 0)),
          pl.BlockSpec((16,), lambda i: (0,)),
          pl.BlockSpec((16, 32), lambda i: (0, 0)),
      ],
      out_specs=pl.BlockSpec((BN, 16), lambda i: (i, 0)),
      out_shape=jax.ShapeDtypeStruct((N, 16), jnp.float32),
  )(p2, h1, w_rel, b, w_root)


def _final(p3, h2, b3, w_rel, w_root):
  def body(p_ref, h_ref, b_ref, wr_ref, wt_ref, o_ref):
    pb = p_ref[...]
    agg = pb[0] + pb[1]
    logits = (_dot_t(agg, wr_ref[...]) + b_ref[...][None, :]
              + _dot_t(h_ref[...], wt_ref[...]))
    mx = jnp.max(logits, axis=-1, keepdims=True)
    ex = jnp.exp(logits - mx)
    o_ref[...] = ex / jnp.sum(ex, axis=-1, keepdims=True)

  return pl.pallas_call(
      body,
      grid=(NBLK,),
      in_specs=[
          pl.BlockSpec((2, BN, 16), lambda i: (0, i, 0)),
          pl.BlockSpec((BN, 16), lambda i: (i, 0)),
          pl.BlockSpec((2,), lambda i: (0,)),
          pl.BlockSpec((2, 16), lambda i: (0, 0)),
          pl.BlockSpec((2, 16), lambda i: (0, 0)),
      ],
      out_specs=pl.BlockSpec((BN, 2), lambda i: (i, 0)),
      out_shape=jax.ShapeDtypeStruct((N, 2), jnp.float32),
  )(p3, h2, b3, w_rel, w_root)


def kernel(z, edge_index, W1_rel, b1, W1_root, W2_rel, b2, W2_root,
           W3_rel, b3, W3_root):
  pad = EP - E
  src = jnp.concatenate(
      [edge_index[0], jnp.zeros((pad,), jnp.int32)]).reshape(-1, CH)
  dst = jnp.concatenate(
      [edge_index[1], jnp.full((pad,), N, jnp.int32)]).reshape(-1, CH)

  z_lo, z_hi = _split(z)
  p1 = _segsum_cols(src, dst, z_lo, z_hi, 32)
  h1 = _layer1(p1, z, W1_rel, b1, W1_root)
  p2 = _segsum_edges(src, dst, h1, 32)
  h2 = _layer2(p2, h1, W2_rel, b2, W2_root)
  p3 = _segsum_edges(src, dst, h2, 16)
  return _final(p3, h2, b3, W3_rel, W3_root)


# speedup vs baseline: 8.5241x; 1.3194x over previous
import functools

import jax
import jax.numpy as jnp
from jax import lax
from jax.experimental import pallas as pl
from jax.experimental.pallas import tpu as pltpu
from jax.experimental.pallas import tpu_sc as plsc

N = 50000
E = 800000
NC = 2
NS = 16
NW = NC * NS
CH = 128
CHUNKS_W = 200
EP = NW * CH * CHUNKS_W
CHUNKS_T = EP // (CH * NS)
N_PAD = 50176
G = 40
BN = 2000
NBLK = N // BN


NBUF = 4


def _seg_body(feat_hbm, src_hbm, dst_hbm, sidx, didx, bufs, acc, semg, sems,
              chunk0, nchunks):
  def gather_start(jj, t):
    pltpu.async_copy(feat_hbm.at[sidx.at[jj]], bufs.at[t], semg.at[t])

  def gather_wait(jj, t):
    pltpu.make_async_copy(feat_hbm.at[sidx.at[jj]], bufs.at[t],
                          semg.at[t]).wait()

  def scatter_start(jj, t):
    pltpu.async_copy(bufs.at[t], acc.at[didx.at[jj]], sems.at[t], add=True)

  def scatter_wait(jj, t):
    pltpu.make_async_copy(bufs.at[t], acc.at[didx.at[jj]], sems.at[t]).wait()

  def group(gi, carry):
    base = chunk0 + gi * G
    pltpu.sync_copy(src_hbm.at[pl.ds(base, G)], sidx)
    pltpu.sync_copy(dst_hbm.at[pl.ds(base, G)], didx)
    for t in range(NBUF):
      gather_start(t, t)

    def body(k, carry2):
      j0 = k * NBUF
      for t in range(NBUF):
        gather_wait(j0 + t, t)
        scatter_start(j0 + t, t)
      for t in range(NBUF):
        scatter_wait(j0 + t, t)

        @pl.when(j0 + NBUF + t < G)
        def _():
          gather_start(j0 + NBUF + t, t)

      return carry2

    return lax.fori_loop(0, G // NBUF, body, carry)

  lax.fori_loop(0, nchunks // G, group, 0)


def _zero_acc(rows, acc, s, d):
  z16 = jnp.zeros((16,), jnp.float32)

  def zrow(i, carry):
    for g in range(d // 16):
      rows[i, pl.ds(g * 16, 16)] = z16
    return carry

  lax.fori_loop(0, CH, zrow, 0)
  zbase = s * (N_PAD // NS)

  def zcopy(j, carry):
    pltpu.sync_copy(rows, acc.at[pl.ds(zbase + j * CH, CH)])
    return carry

  lax.fori_loop(0, (N_PAD // NS) // CH, zcopy, 0)
  if (N_PAD // NS) % CH:
    pltpu.sync_copy(rows, acc.at[pl.ds(zbase + (N_PAD // NS) - CH, CH)])


def _writeout(acc, out_hbm, c, s):
  wrows = N_PAD // NS
  wbase = s * wrows
  pltpu.sync_copy(acc.at[pl.ds(wbase, wrows)],
                  out_hbm.at[c, pl.ds(wbase, wrows)])


def _scratch(d):
  return [
      pltpu.VMEM((G, CH), jnp.int32),
      pltpu.VMEM((G, CH), jnp.int32),
      pltpu.VMEM((NBUF, CH, d), jnp.float32),
      pltpu.VMEM_SHARED((N_PAD, d), jnp.float32),
      pltpu.SemaphoreType.DMA((NBUF,)),
      pltpu.SemaphoreType.DMA((NBUF,)),
  ]


def _mesh():
  return plsc.VectorSubcoreMesh(
      core_axis_name="c", subcore_axis_name="s", num_cores=NC, num_subcores=NS)


def _segsum_cols(src2d, dst2d, feat_lo, feat_hi, d):
  @functools.partial(
      pl.kernel,
      out_type=jax.ShapeDtypeStruct((NC, N_PAD, d), jnp.float32),
      mesh=_mesh(),
      compiler_params=pltpu.CompilerParams(use_tc_tiling_on_sc=False),
      scratch_types=_scratch(d),
  )
  def seg(src_hbm, dst_hbm, lo_hbm, hi_hbm, out_hbm, sidx, didx, bufs, acc,
          semg, sems):
    c = lax.axis_index("c")
    s = lax.axis_index("s")
    _zero_acc(bufs.at[0], acc, s, d)
    plsc.subcore_barrier()

    @pl.when(c == 0)
    def _():
      _seg_body(lo_hbm, src_hbm, dst_hbm, sidx, didx, bufs, acc, semg, sems,
                s * CHUNKS_T, CHUNKS_T)

    @pl.when(c == 1)
    def _():
      _seg_body(hi_hbm, src_hbm, dst_hbm, sidx, didx, bufs, acc, semg, sems,
                s * CHUNKS_T, CHUNKS_T)

    plsc.subcore_barrier()
    _writeout(acc, out_hbm, c, s)

  return seg(src2d, dst2d, feat_lo, feat_hi)


def _segsum_edges(src2d, dst2d, feat, d):
  @functools.partial(
      pl.kernel,
      out_type=jax.ShapeDtypeStruct((NC, N_PAD, d), jnp.float32),
      mesh=_mesh(),
      compiler_params=pltpu.CompilerParams(use_tc_tiling_on_sc=False),
      scratch_types=_scratch(d),
  )
  def seg(src_hbm, dst_hbm, feat_hbm, out_hbm, sidx, didx, bufs, acc,
          semg, sems):
    c = lax.axis_index("c")
    s = lax.axis_index("s")
    _zero_acc(bufs.at[0], acc, s, d)
    plsc.subcore_barrier()
    wid = c * NS + s
    _seg_body(feat_hbm, src_hbm, dst_hbm, sidx, didx, bufs, acc, semg, sems,
              wid * CHUNKS_W, CHUNKS_W)
    plsc.subcore_barrier()
    _writeout(acc, out_hbm, c, s)

  return seg(src2d, dst2d, feat)


def _dot_t(x, w):
  return lax.dot_general(x, w, (((1,), (1,)), ((), ())),
                         preferred_element_type=jnp.float32)


def _split(z):
  def body(z_ref, lo_ref, hi_ref):
    zb = z_ref[...]
    lo_ref[...] = zb[:, :32]
    hi_ref[...] = zb[:, 32:]

  sds = jax.ShapeDtypeStruct((N, 32), jnp.float32)
  return pl.pallas_call(
      body,
      grid=(NBLK,),
      in_specs=[pl.BlockSpec((BN, 64), lambda i: (i, 0))],
      out_specs=[
          pl.BlockSpec((BN, 32), lambda i: (i, 0)),
          pl.BlockSpec((BN, 32), lambda i: (i, 0)),
      ],
      out_shape=(sds, sds),
  )(z)


def _layer1(p1, z, w_rel, b, w_root):
  def body(p_ref, z_ref, wr_ref, b_ref, wt_ref, h_ref):
    pb = p_ref[...]
    wr = wr_ref[...]
    agg_dot = _dot_t(pb[0], wr[:, :32]) + _dot_t(pb[1], wr[:, 32:])
    h_ref[...] = jnp.maximum(
        agg_dot + b_ref[...][None, :] + _dot_t(z_ref[...], wt_ref[...]), 0.0)

  return pl.pallas_call(
      body,
      grid=(NBLK,),
      in_specs=[
          pl.BlockSpec((2, BN, 32), lambda i: (0, i, 0)),
          pl.BlockSpec((BN, 64), lambda i: (i, 0)),
          pl.BlockSpec((32, 64), lambda i: (0, 0)),
          pl.BlockSpec((32,), lambda i: (0,)),
          pl.BlockSpec((32, 64), lambda i: (0, 0)),
      ],
      out_specs=pl.BlockSpec((BN, 32), lambda i: (i, 0)),
      out_shape=jax.ShapeDtypeStruct((N, 32), jnp.float32),
  )(p1, z, w_rel, b, w_root)


def _layer2(p2, h1, w_rel, b, w_root):
  def body(p_ref, h_ref, wr_ref, b_ref, wt_ref, o_ref):
    pb = p_ref[...]
    agg = pb[0] + pb[1]
    o_ref[...] = jnp.maximum(
        _dot_t(agg, wr_ref[...]) + b_ref[...][None, :]
        + _dot_t(h_ref[...], wt_ref[...]), 0.0)

  return pl.pallas_call(
      body,
      grid=(NBLK,),
      in_specs=[
          pl.BlockSpec((2, BN, 32), lambda i: (0, i, 0)),
          pl.BlockSpec((BN, 32), lambda i: (i, 0)),
          pl.BlockSpec((16, 32), lambda i: (0, 0)),
          pl.BlockSpec((16,), lambda i: (0,)),
          pl.BlockSpec((16, 32), lambda i: (0, 0)),
      ],
      out_specs=pl.BlockSpec((BN, 16), lambda i: (i, 0)),
      out_shape=jax.ShapeDtypeStruct((N, 16), jnp.float32),
  )(p2, h1, w_rel, b, w_root)


def _final(p3, h2, b3, w_rel, w_root):
  def body(p_ref, h_ref, b_ref, wr_ref, wt_ref, o_ref):
    pb = p_ref[...]
    agg = pb[0] + pb[1]
    logits = (_dot_t(agg, wr_ref[...]) + b_ref[...][None, :]
              + _dot_t(h_ref[...], wt_ref[...]))
    mx = jnp.max(logits, axis=-1, keepdims=True)
    ex = jnp.exp(logits - mx)
    o_ref[...] = ex / jnp.sum(ex, axis=-1, keepdims=True)

  return pl.pallas_call(
      body,
      grid=(NBLK,),
      in_specs=[
          pl.BlockSpec((2, BN, 16), lambda i: (0, i, 0)),
          pl.BlockSpec((BN, 16), lambda i: (i, 0)),
          pl.BlockSpec((2,), lambda i: (0,)),
          pl.BlockSpec((2, 16), lambda i: (0, 0)),
          pl.BlockSpec((2, 16), lambda i: (0, 0)),
      ],
      out_specs=pl.BlockSpec((BN, 2), lambda i: (i, 0)),
      out_shape=jax.ShapeDtypeStruct((N, 2), jnp.float32),
  )(p3, h2, b3, w_rel, w_root)


def kernel(z, edge_index, W1_rel, b1, W1_root, W2_rel, b2, W2_root,
           W3_rel, b3, W3_root):
  pad = EP - E
  src = jnp.concatenate(
      [edge_index[0], jnp.zeros((pad,), jnp.int32)]).reshape(-1, CH)
  dst = jnp.concatenate(
      [edge_index[1],
       N + (jnp.arange(pad, dtype=jnp.int32) % (N_PAD - N))]).reshape(-1, CH)

  z_lo, z_hi = _split(z)
  p1 = _segsum_cols(src, dst, z_lo, z_hi, 32)
  h1 = _layer1(p1, z, W1_rel, b1, W1_root)
  p2 = _segsum_edges(src, dst, h1, 32)
  h2 = _layer2(p2, h1, W2_rel, b2, W2_root)
  p3 = _segsum_edges(src, dst, h2, 16)
  return _final(p3, h2, b3, W3_rel, W3_root)
